# Initial kernel scaffold; baseline (speedup 1.0000x reference)
import jax
import jax.numpy as jnp
from jax.experimental import pallas as pl


def kernel(x, edge_index, edge_attr, valid_node_indices, steps_till_done, ep_length, gat_W0, gat_We0, gat_a0, gat_W1, gat_We1, gat_a1, gat_W2, gat_We2, gat_a2, A_W1, A_b1, A_W2, A_b2, V_W1, V_b1, V_W2, V_b2):
    raise NotImplementedError("write your pallas kernel here")


# speedup vs baseline: 22.5106x; 22.5106x over previous
import functools
import jax
import jax.numpy as jnp
from jax import lax
from jax.experimental import pallas as pl
from jax.experimental.pallas import tpu as pltpu
from jax.experimental.pallas import tpu_sc as plsc

N_NODES = 10000
N_EDGES = 320000
HEADS = 8
HIDDEN = 32
D = HEADS * HIDDEN
ROW = 272
NC, NS, LANES = 2, 16, 16
HALF = N_NODES // NC
ACC_ROWS = 5008
TRASH = 5004
CHUNK = 80
EPT = N_EDGES // NS
NBLK = 400
EBLK = 2000

_i32 = jnp.int32
_f32 = jnp.float32


def _proj_body(first, h_ref, w_ref, a0_ref, a1_ref, tsrc_ref, sdst_ref):
    h = h_ref[...]
    if not first:
        feat = h[:, :D].reshape(NBLK, HEADS, HIDDEN)
        denom = h[:, D:D + HEADS].reshape(NBLK, HEADS, 1)
        hv = (feat / (denom + 1e-16)).reshape(NBLK, D)
        h = jnp.where(hv > 0, hv, jnp.exp(jnp.minimum(hv, 0.0)) - 1.0)
    hp = jnp.dot(h, w_ref[...], preferred_element_type=_f32)
    hp3 = hp.reshape(NBLK, HEADS, HIDDEN)
    s0 = jnp.sum(hp3 * a0_ref[...][None], axis=-1)
    s1 = jnp.sum(hp3 * a1_ref[...][None], axis=-1)
    zpad = jnp.zeros((NBLK, ROW - D - HEADS), _f32)
    tsrc_ref[...] = jnp.concatenate([hp, s0, zpad], axis=1)
    sdst_ref[...] = jnp.concatenate([s1, zpad], axis=1)


def _proj(h, w, a0, a1, first):
    d_in = h.shape[1]
    grid = N_NODES // NBLK if first else (N_NODES * ROW // (NBLK * ROW))
    grid = N_NODES // NBLK
    return pl.pallas_call(
        functools.partial(_proj_body, first),
        grid=(grid,),
        in_specs=[
            pl.BlockSpec((NBLK, d_in), lambda i: (i, 0)),
            pl.BlockSpec((d_in if first else D, D), lambda i: (0, 0)),
            pl.BlockSpec((HEADS, HIDDEN), lambda i: (0, 0)),
            pl.BlockSpec((HEADS, HIDDEN), lambda i: (0, 0)),
        ],
        out_specs=[
            pl.BlockSpec((NBLK, ROW), lambda i: (i, 0)),
            pl.BlockSpec((NBLK, LANES), lambda i: (i, 0)),
        ],
        out_shape=[
            jax.ShapeDtypeStruct((N_NODES, ROW), _f32),
            jax.ShapeDtypeStruct((N_NODES, LANES), _f32),
        ],
    )(h, w, a0, a1)


def _se_body(ea_ref, we0, a20, we1, a21, we2, a22, o0, o1, o2):
    ea = ea_ref[...]
    for we_ref, a_ref, o_ref in ((we0, a20, o0), (we1, a21, o1), (we2, a22, o2)):
        c = jnp.sum(we_ref[...].reshape(16, HEADS, HIDDEN) * a_ref[...][None],
                    axis=-1)
        se = jnp.dot(ea, c, preferred_element_type=_f32)
        o_ref[...] = se.reshape(EBLK // LANES, LANES, HEADS).transpose(0, 2, 1)


def _se_all(edge_attr, we0, a20, we1, a21, we2, a22):
    grid = N_EDGES // EBLK
    rows = EBLK // LANES
    wspec = pl.BlockSpec((16, D), lambda i: (0, 0))
    aspec = pl.BlockSpec((HEADS, HIDDEN), lambda i: (0, 0))
    ospec = pl.BlockSpec((rows, HEADS, LANES), lambda i: (i, 0, 0))
    oshape = jax.ShapeDtypeStruct((N_EDGES // LANES, HEADS, LANES), _f32)
    return pl.pallas_call(
        _se_body,
        grid=(grid,),
        in_specs=[pl.BlockSpec((EBLK, 16), lambda i: (i, 0)),
                  wspec, aspec, wspec, aspec, wspec, aspec],
        out_specs=[ospec, ospec, ospec],
        out_shape=[oshape, oshape, oshape],
    )(edge_attr, we0, a20, we1, a21, we2, a22)


def _edge_body(tsrc, sdst, se, src, dst, out,
               rows_v, sdv, sev, srcv, dstv, dlocv, exbuf, zv, acc, sem):
    c = lax.axis_index("c")
    s = lax.axis_index("s")
    zero16 = jnp.zeros((LANES,), _f32)
    iota16 = lax.iota(_i32, LANES)

    def zrow(r, _):
        for k in range(ROW // LANES):
            zv[r, pl.ds(k * LANES, LANES)] = zero16
        return 0
    lax.fori_loop(0, LANES, zrow, 0)

    def zacc(i, _):
        pltpu.sync_copy(zv, acc.at[pl.ds(s * 312 + i * LANES, LANES)])
        return 0
    lax.fori_loop(0, 19, zacc, 0)
    pltpu.sync_copy(zv.at[pl.ds(0, 8)], acc.at[pl.ds(s * 312 + 304, 8)])
    @pl.when(s == NS - 1)
    def _():
        pltpu.sync_copy(zv, acc.at[pl.ds(4992, LANES)])
    plsc.subcore_barrier()

    half_base = c * HALF

    def chunk(ci, _):
        base = s * EPT + ci * CHUNK
        pltpu.sync_copy(src.at[pl.ds(base, CHUNK)], srcv)
        pltpu.sync_copy(dst.at[pl.ds(base, CHUNK)], dstv)
        pltpu.sync_copy(se.at[pl.ds(s * (EPT // LANES) + ci * (CHUNK // LANES),
                                    CHUNK // LANES)], sev)
        pltpu.async_copy(tsrc.at[srcv], rows_v, sem).wait()
        pltpu.async_copy(sdst.at[dstv], sdv, sem).wait()

        def group(g, _):
            rid = iota16 + g * LANES
            exvecs = []
            for h in range(HEADS):
                ss = plsc.load_gather(rows_v, [rid, jnp.full((LANES,), D + h, _i32)])
                sd = plsc.load_gather(sdv, [rid, jnp.full((LANES,), h, _i32)])
                lg = ss + sd + sev[g, h, :]
                lg = jnp.where(lg > 0, lg, 0.2 * lg)
                ex = jnp.exp(lg)
                exbuf[h, :] = ex
                exvecs.append(ex)
            dvec = dstv[pl.ds(g * LANES, LANES)]
            loc = dvec - half_base
            ok = (loc >= 0) & (loc < HALF)
            dlocv[pl.ds(g * LANES, LANES)] = jnp.where(ok, loc, TRASH)
            for e in range(LANES):
                r = g * LANES + e
                for h in range(HEADS):
                    sc = exvecs[h][e]
                    for q in range(2):
                        sl = pl.ds(h * HIDDEN + q * LANES, LANES)
                        rows_v[r, sl] = rows_v[r, sl] * sc
                exv = plsc.load_gather(
                    exbuf, [iota16 & 7, jnp.full((LANES,), e, _i32)])
                rows_v[r, pl.ds(D, LANES)] = jnp.where(iota16 < HEADS, exv, 0.0)
            return 0

        lax.fori_loop(0, CHUNK // LANES, group, 0)
        pltpu.sync_copy(rows_v, acc.at[dlocv], add=True)
        return 0

    lax.fori_loop(0, EPT // CHUNK, chunk, 0)
    plsc.subcore_barrier()

    pltpu.sync_copy(acc.at[pl.ds(s * 312, 312)],
                    out.at[pl.ds(c * HALF + s * 312, 312)])
    @pl.when(s == NS - 1)
    def _():
        pltpu.sync_copy(acc.at[pl.ds(4992, 8)],
                        out.at[pl.ds(c * HALF + 4992, 8)])


def _sc_edge(tsrc, sdst, se, src, dst):
    mesh = plsc.VectorSubcoreMesh(core_axis_name="c", subcore_axis_name="s")
    f = pl.kernel(
        _edge_body,
        out_type=jax.ShapeDtypeStruct((N_NODES, ROW), _f32),
        mesh=mesh,
        compiler_params=pltpu.CompilerParams(use_tc_tiling_on_sc=False, needs_layout_passes=False),
        scratch_types=[
            pltpu.VMEM((CHUNK, ROW), _f32),
            pltpu.VMEM((CHUNK, LANES), _f32),
            pltpu.VMEM((CHUNK // LANES, HEADS, LANES), _f32),
            pltpu.VMEM((CHUNK,), _i32),
            pltpu.VMEM((CHUNK,), _i32),
            pltpu.VMEM((CHUNK,), _i32),
            pltpu.VMEM((HEADS, LANES), _f32),
            pltpu.VMEM((LANES, ROW), _f32),
            pltpu.VMEM_SHARED((ACC_ROWS, ROW), _f32),
            pltpu.SemaphoreType.DMA,
        ],
    )
    return f(tsrc, sdst, se, src, dst)


def _gather_body(table, idx, out, idx_v, rows_v, sem):
    wid = lax.axis_index("s") * NC + lax.axis_index("c")
    bpw = 1024 // (NC * NS)
    base = wid * bpw
    pltpu.sync_copy(idx.at[pl.ds(base, bpw)], idx_v)
    pltpu.async_copy(table.at[idx_v], rows_v, sem).wait()
    pltpu.sync_copy(rows_v, out.at[pl.ds(base, bpw)])


def _sc_gather(table, idx):
    bpw = 1024 // (NC * NS)
    mesh = plsc.VectorSubcoreMesh(core_axis_name="c", subcore_axis_name="s")
    f = pl.kernel(
        _gather_body,
        out_type=jax.ShapeDtypeStruct((1024, ROW), _f32),
        mesh=mesh,
        compiler_params=pltpu.CompilerParams(use_tc_tiling_on_sc=False,
                                             needs_layout_passes=False),
        scratch_types=[
            pltpu.VMEM((bpw,), _i32),
            pltpu.VMEM((bpw, ROW), _f32),
            pltpu.SemaphoreType.DMA,
        ],
    )
    return f(table, idx)


def _reduce_body(acc_ref, o_ref):
    i = pl.program_id(0)
    a = acc_ref[...]
    feat = a[:, :D].reshape(NBLK, HEADS, HIDDEN)
    denom = a[:, D:D + HEADS].reshape(NBLK, HEADS, 1)
    hv = (feat / (denom + 1e-16)).reshape(NBLK, D)
    h = jnp.where(hv > 0, hv, jnp.exp(jnp.minimum(hv, 0.0)) - 1.0)
    part = jnp.sum(h.reshape(NBLK // 8, 8, D), axis=0)

    @pl.when(i == 0)
    def _():
        o_ref[...] = jnp.zeros((8, D), _f32)
    o_ref[...] += part


def _reduce(acc):
    return pl.pallas_call(
        _reduce_body,
        grid=(N_NODES // NBLK,),
        in_specs=[pl.BlockSpec((NBLK, ROW), lambda i: (i, 0))],
        out_specs=pl.BlockSpec((8, D), lambda i: (0, 0)),
        out_shape=jax.ShapeDtypeStruct((8, D), _f32),
    )(acc)


def _final_body(g_ref, hsum_ref, sb_ref, aw1a, aw1b, aw1c, ab1, aw2, ab2,
                vw1a, vw1c, vb1, vw2, vb2, q_ref):
    grow = g_ref[...]
    feat = grow[:, :D].reshape(1024, HEADS, HIDDEN)
    denom = grow[:, D:D + HEADS].reshape(1024, HEADS, 1)
    hv = (feat / (denom + 1e-16)).reshape(1024, D)
    nef = jnp.where(hv > 0, hv, jnp.exp(jnp.minimum(hv, 0.0)) - 1.0)
    agg = jnp.sum(hsum_ref[...], axis=0, keepdims=True) / N_NODES
    sb = sb_ref[0, 0]

    a1 = jnp.dot(nef, aw1a[...], preferred_element_type=_f32)
    a1 = a1 + jnp.dot(agg, aw1b[...], preferred_element_type=_f32)
    a1 = a1 + sb * aw1c[...] + ab1[...]
    a1 = jnp.maximum(a1, 0.0)
    avals = jnp.dot(a1, aw2[...], preferred_element_type=_f32) + ab2[0, 0]

    v1 = jnp.dot(agg, vw1a[...], preferred_element_type=_f32)
    v1 = v1 + sb * vw1c[...] + vb1[...]
    v1 = jnp.maximum(v1, 0.0)
    v = jnp.dot(v1, vw2[...], preferred_element_type=_f32) + vb2[0, 0]

    q = v[0, 0] + avals - jnp.mean(avals)
    q_ref[...] = q.reshape(8, 128)


def _final(grows, hsum, sb, aw1a, aw1b, aw1c, ab1, aw2, ab2,
           vw1a, vw1c, vb1, vw2, vb2):
    specs = [pl.BlockSpec(x.shape, lambda i: tuple(0 for _ in x.shape))
             for x in (grows, hsum, sb, aw1a, aw1b, aw1c, ab1, aw2, ab2,
                       vw1a, vw1c, vb1, vw2, vb2)]
    return pl.pallas_call(
        _final_body,
        grid=(1,),
        in_specs=specs,
        out_specs=pl.BlockSpec((8, 128), lambda i: (0, 0)),
        out_shape=jax.ShapeDtypeStruct((8, 128), _f32),
    )(grows, hsum, sb, aw1a, aw1b, aw1c, ab1, aw2, ab2,
      vw1a, vw1c, vb1, vw2, vb2)


def kernel(x, edge_index, edge_attr, valid_node_indices, steps_till_done,
           ep_length, gat_W0, gat_We0, gat_a0, gat_W1, gat_We1, gat_a1,
           gat_W2, gat_We2, gat_a2, A_W1, A_b1, A_W2, A_b2,
           V_W1, V_b1, V_W2, V_b2):
    src = edge_index[0].astype(_i32)
    dst = edge_index[1].astype(_i32)

    se0, se1, se2 = _se_all(edge_attr, gat_We0, gat_a0[2], gat_We1,
                            gat_a1[2], gat_We2, gat_a2[2])

    tsrc, sdst = _proj(x, gat_W0, gat_a0[0], gat_a0[1], True)
    acc = _sc_edge(tsrc, sdst, se0, src, dst)
    tsrc, sdst = _proj(acc, gat_W1, gat_a1[0], gat_a1[1], False)
    acc = _sc_edge(tsrc, sdst, se1, src, dst)
    tsrc, sdst = _proj(acc, gat_W2, gat_a2[0], gat_a2[1], False)
    acc = _sc_edge(tsrc, sdst, se2, src, dst)

    hsum = _reduce(acc)
    grows = _sc_gather(acc, valid_node_indices.astype(_i32))

    sb = (steps_till_done / ep_length).reshape(1, 1).astype(_f32)
    q = _final(grows, hsum, sb,
               A_W1[:D], A_W1[D:2 * D], A_W1[2 * D:2 * D + 1],
               A_b1.reshape(1, -1), A_W2, A_b2.reshape(1, 1),
               V_W1[:D], V_W1[D:D + 1], V_b1.reshape(1, -1),
               V_W2, V_b2.reshape(1, 1))
    return q.reshape(1024), valid_node_indices


# speedup vs baseline: 27.0486x; 1.2016x over previous
import functools
import jax
import jax.numpy as jnp
from jax import lax
from jax.experimental import pallas as pl
from jax.experimental.pallas import tpu as pltpu
from jax.experimental.pallas import tpu_sc as plsc

N_NODES = 10000
N_EDGES = 320000
HEADS = 8
HIDDEN = 32
D = HEADS * HIDDEN
NC, NS, LANES = 2, 16, 16
HH = HEADS // NC
HD = HH * HIDDEN
ROW = 144
CHUNK = 80
EPT = N_EDGES // NS
NBLK = 400
EBLK = 2000

_i32 = jnp.int32
_f32 = jnp.float32

_SC_PARAMS = pltpu.CompilerParams(use_tc_tiling_on_sc=False,
                                  needs_layout_passes=False)


def _unsplit(blk):
    feat = jnp.concatenate([blk[0, :, :HD], blk[1, :, :HD]], axis=1)
    denom = jnp.concatenate(
        [blk[0, :, HD:HD + HH], blk[1, :, HD:HD + HH]], axis=1)
    return feat, denom


def _norm_elu(blk):
    n = blk.shape[1]
    feat, denom = _unsplit(blk)
    f3 = feat.reshape(n, HEADS, HIDDEN)
    d3 = denom.reshape(n, HEADS, 1)
    hv = (f3 / (d3 + 1e-16)).reshape(n, D)
    return jnp.where(hv > 0, hv, jnp.exp(jnp.minimum(hv, 0.0)) - 1.0)


def _proj_body(first, h_ref, w_ref, a0_ref, a1_ref, tsrc_ref, sdst_ref):
    if first:
        h = h_ref[...]
    else:
        h = _norm_elu(h_ref[...])
    hp = jnp.dot(h, w_ref[...], preferred_element_type=_f32)
    hp3 = hp.reshape(NBLK, HEADS, HIDDEN)
    s0 = jnp.sum(hp3 * a0_ref[...][None], axis=-1)
    s1 = jnp.sum(hp3 * a1_ref[...][None], axis=-1)
    zpad = jnp.zeros((NBLK, ROW - HD - HH), _f32)
    halves = []
    for c in range(NC):
        halves.append(jnp.concatenate(
            [hp[:, c * HD:(c + 1) * HD], s0[:, c * HH:(c + 1) * HH], zpad],
            axis=1))
    tsrc_ref[...] = jnp.stack(halves, axis=0)
    sdst_ref[...] = jnp.concatenate(
        [s1, jnp.zeros((NBLK, LANES - HEADS), _f32)], axis=1)


def _proj(h, w, a0, a1, first):
    if first:
        hspec = pl.BlockSpec((NBLK, h.shape[1]), lambda i: (i, 0))
        wrows = h.shape[1]
    else:
        hspec = pl.BlockSpec((NC, NBLK, ROW), lambda i: (0, i, 0))
        wrows = D
    return pl.pallas_call(
        functools.partial(_proj_body, first),
        grid=(N_NODES // NBLK,),
        in_specs=[
            hspec,
            pl.BlockSpec((wrows, D), lambda i: (0, 0)),
            pl.BlockSpec((HEADS, HIDDEN), lambda i: (0, 0)),
            pl.BlockSpec((HEADS, HIDDEN), lambda i: (0, 0)),
        ],
        out_specs=[
            pl.BlockSpec((NC, NBLK, ROW), lambda i: (0, i, 0)),
            pl.BlockSpec((NBLK, LANES), lambda i: (i, 0)),
        ],
        out_shape=[
            jax.ShapeDtypeStruct((NC, N_NODES, ROW), _f32),
            jax.ShapeDtypeStruct((N_NODES, LANES), _f32),
        ],
    )(h, w, a0, a1)


def _se_body(ea_ref, we0, a20, we1, a21, we2, a22, o0, o1, o2):
    ea = ea_ref[...]
    for we_ref, a_ref, o_ref in ((we0, a20, o0), (we1, a21, o1), (we2, a22, o2)):
        c = jnp.sum(we_ref[...].reshape(16, HEADS, HIDDEN) * a_ref[...][None],
                    axis=-1)
        se = jnp.dot(ea, c, preferred_element_type=_f32)
        slabs = []
        for cc in range(NC):
            sec = se[:, cc * HH:(cc + 1) * HH]
            slabs.append(sec.reshape(EBLK // LANES, LANES, HH).transpose(0, 2, 1))
        o_ref[...] = jnp.stack(slabs, axis=0)


def _se_all(edge_attr, we0, a20, we1, a21, we2, a22):
    rows = EBLK // LANES
    wspec = pl.BlockSpec((16, D), lambda i: (0, 0))
    aspec = pl.BlockSpec((HEADS, HIDDEN), lambda i: (0, 0))
    ospec = pl.BlockSpec((NC, rows, HH, LANES), lambda i: (0, i, 0, 0))
    oshape = jax.ShapeDtypeStruct((NC, N_EDGES // LANES, HH, LANES), _f32)
    return pl.pallas_call(
        _se_body,
        grid=(N_EDGES // EBLK,),
        in_specs=[pl.BlockSpec((EBLK, 16), lambda i: (i, 0)),
                  wspec, aspec, wspec, aspec, wspec, aspec],
        out_specs=[ospec, ospec, ospec],
        out_shape=[oshape, oshape, oshape],
    )(edge_attr, we0, a20, we1, a21, we2, a22)


def _edge_body(tsrc, sdst, se, src, dst, out,
               rows_v, sdv, sev, srcv, dstv, exbuf, zv, acc, sem):
    c = lax.axis_index("c")
    s = lax.axis_index("s")
    zero16 = jnp.zeros((LANES,), _f32)
    iota16 = lax.iota(_i32, LANES)

    def zrow(r, _):
        for k in range(ROW // LANES):
            zv[r, pl.ds(k * LANES, LANES)] = zero16
        return 0
    lax.fori_loop(0, LANES, zrow, 0)

    def zacc(i, _):
        pltpu.sync_copy(zv, acc.at[pl.ds(s * 624 + i * LANES, LANES)])
        return 0
    lax.fori_loop(0, 39, zacc, 0)
    @pl.when(s == NS - 1)
    def _():
        pltpu.sync_copy(zv, acc.at[pl.ds(9984, LANES)])
    plsc.subcore_barrier()

    src_off = c * N_NODES

    def chunk(ci, _):
        base = s * EPT + ci * CHUNK
        pltpu.sync_copy(src.at[pl.ds(base, CHUNK)], srcv)
        pltpu.sync_copy(dst.at[pl.ds(base, CHUNK)], dstv)
        pltpu.sync_copy(
            se.at[c].at[pl.ds(s * (EPT // LANES) + ci * (CHUNK // LANES),
                              CHUNK // LANES)], sev)
        for g in range(CHUNK // LANES):
            sl = pl.ds(g * LANES, LANES)
            srcv[sl] = srcv[sl] + src_off
        pltpu.async_copy(tsrc.at[srcv], rows_v, sem).wait()
        pltpu.async_copy(sdst.at[dstv], sdv, sem).wait()

        def group(g, _):
            rid = iota16 + g * LANES
            exvecs = []
            for h in range(HH):
                ss = plsc.load_gather(rows_v, [rid, jnp.full((LANES,), HD + h, _i32)])
                sd = plsc.load_gather(sdv, [rid, jnp.full((LANES,), h, _i32) + c * HH])
                lg = ss + sd + sev[g, h, :]
                lg = jnp.where(lg > 0, lg, 0.2 * lg)
                ex = jnp.exp(lg)
                exbuf[h, :] = ex
                exvecs.append(ex)
            for e in range(LANES):
                r = g * LANES + e
                for h in range(HH):
                    sc = exvecs[h][e]
                    for q in range(HIDDEN // LANES):
                        fsl = pl.ds(h * HIDDEN + q * LANES, LANES)
                        rows_v[r, fsl] = rows_v[r, fsl] * sc
                exv = plsc.load_gather(
                    exbuf, [iota16 & 3, jnp.full((LANES,), e, _i32)])
                rows_v[r, pl.ds(HD, LANES)] = jnp.where(iota16 < HH, exv, 0.0)
            return 0

        lax.fori_loop(0, CHUNK // LANES, group, 0)
        pltpu.sync_copy(rows_v, acc.at[dstv], add=True)
        return 0

    lax.fori_loop(0, EPT // CHUNK, chunk, 0)
    plsc.subcore_barrier()

    pltpu.sync_copy(acc.at[pl.ds(s * 624, 624)],
                    out.at[c].at[pl.ds(s * 624, 624)])
    @pl.when(s == NS - 1)
    def _():
        pltpu.sync_copy(acc.at[pl.ds(9984, LANES)],
                        out.at[c].at[pl.ds(9984, LANES)])


def _sc_edge(tsrc, sdst, se, src, dst):
    mesh = plsc.VectorSubcoreMesh(core_axis_name="c", subcore_axis_name="s")
    f = pl.kernel(
        _edge_body,
        out_type=jax.ShapeDtypeStruct((NC, N_NODES, ROW), _f32),
        mesh=mesh,
        compiler_params=_SC_PARAMS,
        scratch_types=[
            pltpu.VMEM((CHUNK, ROW), _f32),
            pltpu.VMEM((CHUNK, LANES), _f32),
            pltpu.VMEM((CHUNK // LANES, HH, LANES), _f32),
            pltpu.VMEM((CHUNK,), _i32),
            pltpu.VMEM((CHUNK,), _i32),
            pltpu.VMEM((HH, LANES), _f32),
            pltpu.VMEM((LANES, ROW), _f32),
            pltpu.VMEM_SHARED((N_NODES, ROW), _f32),
            pltpu.SemaphoreType.DMA,
        ],
    )
    return f(tsrc.reshape(NC * N_NODES, ROW), sdst, se, src, dst)


def _gather_body(table, idx, out, idx_v, rows_v, sem):
    c = lax.axis_index("c")
    s = lax.axis_index("s")
    wid = s * NC + c
    bpw = 1024 // (NC * NS)
    base = wid * bpw
    pltpu.sync_copy(idx.at[pl.ds(base, bpw)], idx_v)
    for half in range(NC):
        pltpu.async_copy(table.at[half].at[idx_v], rows_v, sem).wait()
        pltpu.sync_copy(rows_v, out.at[half].at[pl.ds(base, bpw)])


def _sc_gather(table, idx):
    bpw = 1024 // (NC * NS)
    mesh = plsc.VectorSubcoreMesh(core_axis_name="c", subcore_axis_name="s")
    f = pl.kernel(
        _gather_body,
        out_type=jax.ShapeDtypeStruct((NC, 1024, ROW), _f32),
        mesh=mesh,
        compiler_params=_SC_PARAMS,
        scratch_types=[
            pltpu.VMEM((bpw,), _i32),
            pltpu.VMEM((bpw, ROW), _f32),
            pltpu.SemaphoreType.DMA,
        ],
    )
    return f(table, idx)


def _reduce_body(acc_ref, o_ref):
    i = pl.program_id(0)
    h = _norm_elu(acc_ref[...])
    part = jnp.sum(h.reshape(NBLK // 8, 8, D), axis=0)

    @pl.when(i == 0)
    def _():
        o_ref[...] = jnp.zeros((8, D), _f32)
    o_ref[...] += part


def _reduce(acc):
    return pl.pallas_call(
        _reduce_body,
        grid=(N_NODES // NBLK,),
        in_specs=[pl.BlockSpec((NC, NBLK, ROW), lambda i: (0, i, 0))],
        out_specs=pl.BlockSpec((8, D), lambda i: (0, 0)),
        out_shape=jax.ShapeDtypeStruct((8, D), _f32),
    )(acc)


def _final_body(g_ref, hsum_ref, sb_ref, aw1a, aw1b, aw1c, ab1, aw2, ab2,
                vw1a, vw1c, vb1, vw2, vb2, q_ref):
    nef = _norm_elu(g_ref[...])
    agg = jnp.sum(hsum_ref[...], axis=0, keepdims=True) / N_NODES
    sb = sb_ref[0, 0]

    a1 = jnp.dot(nef, aw1a[...], preferred_element_type=_f32)
    a1 = a1 + jnp.dot(agg, aw1b[...], preferred_element_type=_f32)
    a1 = a1 + sb * aw1c[...] + ab1[...]
    a1 = jnp.maximum(a1, 0.0)
    avals = jnp.dot(a1, aw2[...], preferred_element_type=_f32) + ab2[0, 0]

    v1 = jnp.dot(agg, vw1a[...], preferred_element_type=_f32)
    v1 = v1 + sb * vw1c[...] + vb1[...]
    v1 = jnp.maximum(v1, 0.0)
    v = jnp.dot(v1, vw2[...], preferred_element_type=_f32) + vb2[0, 0]

    q = v[0, 0] + avals - jnp.mean(avals)
    q_ref[...] = q.reshape(8, 128)


def _final(grows, hsum, sb, aw1a, aw1b, aw1c, ab1, aw2, ab2,
           vw1a, vw1c, vb1, vw2, vb2):
    args = (grows, hsum, sb, aw1a, aw1b, aw1c, ab1, aw2, ab2,
            vw1a, vw1c, vb1, vw2, vb2)
    specs = []
    for x in args:
        nd = len(x.shape)
        specs.append(pl.BlockSpec(x.shape, (lambda i: (0, 0, 0)) if nd == 3
                                  else (lambda i: (0, 0))))
    return pl.pallas_call(
        _final_body,
        grid=(1,),
        in_specs=specs,
        out_specs=pl.BlockSpec((8, 128), lambda i: (0, 0)),
        out_shape=jax.ShapeDtypeStruct((8, 128), _f32),
    )(*args)


def kernel(x, edge_index, edge_attr, valid_node_indices, steps_till_done,
           ep_length, gat_W0, gat_We0, gat_a0, gat_W1, gat_We1, gat_a1,
           gat_W2, gat_We2, gat_a2, A_W1, A_b1, A_W2, A_b2,
           V_W1, V_b1, V_W2, V_b2):
    src = edge_index[0].astype(_i32)
    dst = edge_index[1].astype(_i32)

    se0, se1, se2 = _se_all(edge_attr, gat_We0, gat_a0[2], gat_We1,
                            gat_a1[2], gat_We2, gat_a2[2])

    tsrc, sdst = _proj(x, gat_W0, gat_a0[0], gat_a0[1], True)
    acc = _sc_edge(tsrc, sdst, se0, src, dst)
    tsrc, sdst = _proj(acc, gat_W1, gat_a1[0], gat_a1[1], False)
    acc = _sc_edge(tsrc, sdst, se1, src, dst)
    tsrc, sdst = _proj(acc, gat_W2, gat_a2[0], gat_a2[1], False)
    acc = _sc_edge(tsrc, sdst, se2, src, dst)

    hsum = _reduce(acc)
    grows = _sc_gather(acc, valid_node_indices.astype(_i32))

    sb = (steps_till_done / ep_length).reshape(1, 1).astype(_f32)
    q = _final(grows, hsum, sb,
               A_W1[:D], A_W1[D:2 * D], A_W1[2 * D:2 * D + 1],
               A_b1.reshape(1, -1), A_W2, A_b2.reshape(1, 1),
               V_W1[:D], V_W1[D:D + 1], V_b1.reshape(1, -1),
               V_W2, V_b2.reshape(1, 1))
    return q.reshape(1024), valid_node_indices


# speedup vs baseline: 45.8808x; 1.6962x over previous
import functools
import jax
import jax.numpy as jnp
from jax import lax
from jax.experimental import pallas as pl
from jax.experimental.pallas import tpu as pltpu
from jax.experimental.pallas import tpu_sc as plsc

N_NODES = 10000
N_EDGES = 320000
HEADS = 8
HIDDEN = 32
D = HEADS * HIDDEN
NC, NS, LANES = 2, 16, 16
HH = HEADS // NC
HD = HH * HIDDEN
ROW = 144
CHUNK = 80
EPT = N_EDGES // NS
NBLK = 400
EBLK = 2000

_i32 = jnp.int32
_f32 = jnp.float32

_SC_PARAMS = pltpu.CompilerParams(use_tc_tiling_on_sc=False,
                                  needs_layout_passes=False)


def _unsplit(blk):
    feat = jnp.concatenate([blk[0, :, :HD], blk[1, :, :HD]], axis=1)
    denom = jnp.concatenate(
        [blk[0, :, HD:HD + HH], blk[1, :, HD:HD + HH]], axis=1)
    return feat, denom


def _norm_elu(blk):
    n = blk.shape[1]
    feat, denom = _unsplit(blk)
    f3 = feat.reshape(n, HEADS, HIDDEN)
    d3 = denom.reshape(n, HEADS, 1)
    hv = (f3 / (d3 + 1e-16)).reshape(n, D)
    return jnp.where(hv > 0, hv, jnp.exp(jnp.minimum(hv, 0.0)) - 1.0)


def _proj_body(first, h_ref, w_ref, a0_ref, a1_ref, tsrc_ref, sdst_ref):
    if first:
        h = h_ref[...]
    else:
        h = _norm_elu(h_ref[...])
    hp = jnp.dot(h, w_ref[...], preferred_element_type=_f32)
    hp3 = hp.reshape(NBLK, HEADS, HIDDEN)
    s0 = jnp.sum(hp3 * a0_ref[...][None], axis=-1)
    s1 = jnp.sum(hp3 * a1_ref[...][None], axis=-1)
    zpad = jnp.zeros((NBLK, ROW - HD - HH), _f32)
    halves = []
    for c in range(NC):
        halves.append(jnp.concatenate(
            [hp[:, c * HD:(c + 1) * HD], s0[:, c * HH:(c + 1) * HH], zpad],
            axis=1))
    tsrc_ref[...] = jnp.stack(halves, axis=0)
    sdst_ref[...] = jnp.concatenate(
        [s1, jnp.zeros((NBLK, LANES - HEADS), _f32)], axis=1)


def _proj(h, w, a0, a1, first):
    if first:
        hspec = pl.BlockSpec((NBLK, h.shape[1]), lambda i: (i, 0))
        wrows = h.shape[1]
    else:
        hspec = pl.BlockSpec((NC, NBLK, ROW), lambda i: (0, i, 0))
        wrows = D
    return pl.pallas_call(
        functools.partial(_proj_body, first),
        grid=(N_NODES // NBLK,),
        in_specs=[
            hspec,
            pl.BlockSpec((wrows, D), lambda i: (0, 0)),
            pl.BlockSpec((HEADS, HIDDEN), lambda i: (0, 0)),
            pl.BlockSpec((HEADS, HIDDEN), lambda i: (0, 0)),
        ],
        out_specs=[
            pl.BlockSpec((NC, NBLK, ROW), lambda i: (0, i, 0)),
            pl.BlockSpec((NBLK, LANES), lambda i: (i, 0)),
        ],
        out_shape=[
            jax.ShapeDtypeStruct((NC, N_NODES, ROW), _f32),
            jax.ShapeDtypeStruct((N_NODES, LANES), _f32),
        ],
    )(h, w, a0, a1)


def _se_body(ea_ref, we0, a20, we1, a21, we2, a22, o0, o1, o2):
    ea = ea_ref[...]
    for we_ref, a_ref, o_ref in ((we0, a20, o0), (we1, a21, o1), (we2, a22, o2)):
        c = jnp.sum(we_ref[...].reshape(16, HEADS, HIDDEN) * a_ref[...][None],
                    axis=-1)
        se = jnp.dot(ea, c, preferred_element_type=_f32)
        slabs = []
        for cc in range(NC):
            sec = se[:, cc * HH:(cc + 1) * HH]
            slabs.append(sec.reshape(EBLK // LANES, LANES, HH).transpose(0, 2, 1))
        o_ref[...] = jnp.stack(slabs, axis=0)


def _se_all(edge_attr, we0, a20, we1, a21, we2, a22):
    rows = EBLK // LANES
    wspec = pl.BlockSpec((16, D), lambda i: (0, 0))
    aspec = pl.BlockSpec((HEADS, HIDDEN), lambda i: (0, 0))
    ospec = pl.BlockSpec((NC, rows, HH, LANES), lambda i: (0, i, 0, 0))
    oshape = jax.ShapeDtypeStruct((NC, N_EDGES // LANES, HH, LANES), _f32)
    return pl.pallas_call(
        _se_body,
        grid=(N_EDGES // EBLK,),
        in_specs=[pl.BlockSpec((EBLK, 16), lambda i: (i, 0)),
                  wspec, aspec, wspec, aspec, wspec, aspec],
        out_specs=[ospec, ospec, ospec],
        out_shape=[oshape, oshape, oshape],
    )(edge_attr, we0, a20, we1, a21, we2, a22)


def _edge_body(tsrc, sdst, se, src, dst, out,
               rows0, rows1, sdv0, sdv1, sev0, sev1, srcv0, srcv1,
               dstv0, dstv1, exbuf, zv, acc, psem0, psem1, gsem0, gsem1):
    c = lax.axis_index("c")
    s = lax.axis_index("s")
    zero16 = jnp.zeros((LANES,), _f32)
    iota16 = lax.iota(_i32, LANES)
    rows = (rows0, rows1)
    sdvs = (sdv0, sdv1)
    sevs = (sev0, sev1)
    srcvs = (srcv0, srcv1)
    dstvs = (dstv0, dstv1)
    psems = (psem0, psem1)
    gsems = (gsem0, gsem1)

    def zrow(r, _):
        for k in range(ROW // LANES):
            zv[r, pl.ds(k * LANES, LANES)] = zero16
        return 0
    lax.fori_loop(0, LANES, zrow, 0)

    def zacc(i, _):
        pltpu.sync_copy(zv, acc.at[pl.ds(s * 624 + i * LANES, LANES)])
        return 0
    lax.fori_loop(0, 39, zacc, 0)
    @pl.when(s == NS - 1)
    def _():
        pltpu.sync_copy(zv, acc.at[pl.ds(9984, LANES)])
    plsc.subcore_barrier()

    src_off = c * N_NODES

    def stage1(j, ci):
        base = s * EPT + ci * CHUNK
        pltpu.async_copy(src.at[pl.ds(base, CHUNK)], srcvs[j], psems[j])
        pltpu.async_copy(dst.at[pl.ds(base, CHUNK)], dstvs[j], psems[j])
        pltpu.async_copy(
            se.at[c].at[pl.ds(s * (EPT // LANES) + ci * (CHUNK // LANES),
                              CHUNK // LANES)], sevs[j], psems[j])

    def stage2(j):
        pltpu.make_async_copy(src.at[pl.ds(0, CHUNK)], srcvs[j], psems[j]).wait()
        pltpu.make_async_copy(dst.at[pl.ds(0, CHUNK)], dstvs[j], psems[j]).wait()
        pltpu.make_async_copy(
            se.at[c].at[pl.ds(0, CHUNK // LANES)], sevs[j], psems[j]).wait()
        for g in range(CHUNK // LANES):
            sl = pl.ds(g * LANES, LANES)
            srcvs[j][sl] = srcvs[j][sl] + src_off
        pltpu.async_copy(tsrc.at[srcvs[j]], rows[j], gsems[j])
        pltpu.async_copy(sdst.at[dstvs[j]], sdvs[j], gsems[j])

    def waitg(j):
        pltpu.make_async_copy(tsrc.at[srcvs[j]], rows[j], gsems[j]).wait()
        pltpu.make_async_copy(sdst.at[dstvs[j]], sdvs[j], gsems[j]).wait()

    def compute(j):
        rows_v = rows[j]
        sdv = sdvs[j]
        sev = sevs[j]
        dstv = dstvs[j]

        def group(g, _):
            rid = iota16 + g * LANES
            exvecs = []
            for h in range(HH):
                ss = plsc.load_gather(rows_v, [rid, jnp.full((LANES,), HD + h, _i32)])
                sd = plsc.load_gather(sdv, [rid, jnp.full((LANES,), h, _i32) + c * HH])
                lg = ss + sd + sev[g, h, :]
                lg = jnp.where(lg > 0, lg, 0.2 * lg)
                ex = jnp.exp(lg)
                exbuf[h, :] = ex
                exvecs.append(ex)
            for e in range(LANES):
                r = g * LANES + e
                for h in range(HH):
                    sc = exvecs[h][e]
                    for q in range(HIDDEN // LANES):
                        fsl = pl.ds(h * HIDDEN + q * LANES, LANES)
                        rows_v[r, fsl] = rows_v[r, fsl] * sc
                exv = plsc.load_gather(
                    exbuf, [iota16 & 3, jnp.full((LANES,), e, _i32)])
                rows_v[r, pl.ds(HD, LANES)] = jnp.where(iota16 < HH, exv, 0.0)
            return 0

        lax.fori_loop(0, CHUNK // LANES, group, 0)
        pltpu.sync_copy(rows_v, acc.at[dstv], add=True)

    stage1(0, 0)
    stage2(0)
    stage1(1, 1)
    npairs = EPT // CHUNK // 2

    def pair(k, _):
        a = 2 * k
        waitg(0)
        stage2(1)
        compute(0)
        @pl.when(k < npairs - 1)
        def _():
            stage1(0, a + 2)
        waitg(1)
        @pl.when(k < npairs - 1)
        def _():
            stage2(0)
        compute(1)
        @pl.when(k < npairs - 1)
        def _():
            stage1(1, a + 3)
        return 0

    lax.fori_loop(0, npairs, pair, 0)
    plsc.subcore_barrier()

    pltpu.sync_copy(acc.at[pl.ds(s * 624, 624)],
                    out.at[c].at[pl.ds(s * 624, 624)])
    @pl.when(s == NS - 1)
    def _():
        pltpu.sync_copy(acc.at[pl.ds(9984, LANES)],
                        out.at[c].at[pl.ds(9984, LANES)])


def _sc_edge(tsrc, sdst, se, src, dst):
    mesh = plsc.VectorSubcoreMesh(core_axis_name="c", subcore_axis_name="s")
    f = pl.kernel(
        _edge_body,
        out_type=jax.ShapeDtypeStruct((NC, N_NODES, ROW), _f32),
        mesh=mesh,
        compiler_params=_SC_PARAMS,
        scratch_types=[
            pltpu.VMEM((CHUNK, ROW), _f32),
            pltpu.VMEM((CHUNK, ROW), _f32),
            pltpu.VMEM((CHUNK, LANES), _f32),
            pltpu.VMEM((CHUNK, LANES), _f32),
            pltpu.VMEM((CHUNK // LANES, HH, LANES), _f32),
            pltpu.VMEM((CHUNK // LANES, HH, LANES), _f32),
            pltpu.VMEM((CHUNK,), _i32),
            pltpu.VMEM((CHUNK,), _i32),
            pltpu.VMEM((CHUNK,), _i32),
            pltpu.VMEM((CHUNK,), _i32),
            pltpu.VMEM((HH, LANES), _f32),
            pltpu.VMEM((LANES, ROW), _f32),
            pltpu.VMEM_SHARED((N_NODES, ROW), _f32),
            pltpu.SemaphoreType.DMA,
            pltpu.SemaphoreType.DMA,
            pltpu.SemaphoreType.DMA,
            pltpu.SemaphoreType.DMA,
        ],
    )
    return f(tsrc.reshape(NC * N_NODES, ROW), sdst, se, src, dst)


def _gather_body(table, idx, out, idx_v, rows_v, sem):
    c = lax.axis_index("c")
    s = lax.axis_index("s")
    wid = s * NC + c
    bpw = 1024 // (NC * NS)
    base = wid * bpw
    pltpu.sync_copy(idx.at[pl.ds(base, bpw)], idx_v)
    for half in range(NC):
        pltpu.async_copy(table.at[half].at[idx_v], rows_v, sem).wait()
        pltpu.sync_copy(rows_v, out.at[half].at[pl.ds(base, bpw)])


def _sc_gather(table, idx):
    bpw = 1024 // (NC * NS)
    mesh = plsc.VectorSubcoreMesh(core_axis_name="c", subcore_axis_name="s")
    f = pl.kernel(
        _gather_body,
        out_type=jax.ShapeDtypeStruct((NC, 1024, ROW), _f32),
        mesh=mesh,
        compiler_params=_SC_PARAMS,
        scratch_types=[
            pltpu.VMEM((bpw,), _i32),
            pltpu.VMEM((bpw, ROW), _f32),
            pltpu.SemaphoreType.DMA,
        ],
    )
    return f(table, idx)


def _reduce_body(acc_ref, o_ref):
    i = pl.program_id(0)
    h = _norm_elu(acc_ref[...])
    part = jnp.sum(h.reshape(NBLK // 8, 8, D), axis=0)

    @pl.when(i == 0)
    def _():
        o_ref[...] = jnp.zeros((8, D), _f32)
    o_ref[...] += part


def _reduce(acc):
    return pl.pallas_call(
        _reduce_body,
        grid=(N_NODES // NBLK,),
        in_specs=[pl.BlockSpec((NC, NBLK, ROW), lambda i: (0, i, 0))],
        out_specs=pl.BlockSpec((8, D), lambda i: (0, 0)),
        out_shape=jax.ShapeDtypeStruct((8, D), _f32),
    )(acc)


def _final_body(g_ref, hsum_ref, sb_ref, aw1a, aw1b, aw1c, ab1, aw2, ab2,
                vw1a, vw1c, vb1, vw2, vb2, q_ref):
    nef = _norm_elu(g_ref[...])
    agg = jnp.sum(hsum_ref[...], axis=0, keepdims=True) / N_NODES
    sb = sb_ref[0, 0]

    a1 = jnp.dot(nef, aw1a[...], preferred_element_type=_f32)
    a1 = a1 + jnp.dot(agg, aw1b[...], preferred_element_type=_f32)
    a1 = a1 + sb * aw1c[...] + ab1[...]
    a1 = jnp.maximum(a1, 0.0)
    avals = jnp.dot(a1, aw2[...], preferred_element_type=_f32) + ab2[0, 0]

    v1 = jnp.dot(agg, vw1a[...], preferred_element_type=_f32)
    v1 = v1 + sb * vw1c[...] + vb1[...]
    v1 = jnp.maximum(v1, 0.0)
    v = jnp.dot(v1, vw2[...], preferred_element_type=_f32) + vb2[0, 0]

    q = v[0, 0] + avals - jnp.mean(avals)
    q_ref[...] = q.reshape(8, 128)


def _final(grows, hsum, sb, aw1a, aw1b, aw1c, ab1, aw2, ab2,
           vw1a, vw1c, vb1, vw2, vb2):
    args = (grows, hsum, sb, aw1a, aw1b, aw1c, ab1, aw2, ab2,
            vw1a, vw1c, vb1, vw2, vb2)
    specs = []
    for x in args:
        nd = len(x.shape)
        specs.append(pl.BlockSpec(x.shape, (lambda i: (0, 0, 0)) if nd == 3
                                  else (lambda i: (0, 0))))
    return pl.pallas_call(
        _final_body,
        grid=(1,),
        in_specs=specs,
        out_specs=pl.BlockSpec((8, 128), lambda i: (0, 0)),
        out_shape=jax.ShapeDtypeStruct((8, 128), _f32),
    )(*args)


def kernel(x, edge_index, edge_attr, valid_node_indices, steps_till_done,
           ep_length, gat_W0, gat_We0, gat_a0, gat_W1, gat_We1, gat_a1,
           gat_W2, gat_We2, gat_a2, A_W1, A_b1, A_W2, A_b2,
           V_W1, V_b1, V_W2, V_b2):
    src = edge_index[0].astype(_i32)
    dst = edge_index[1].astype(_i32)

    se0, se1, se2 = _se_all(edge_attr, gat_We0, gat_a0[2], gat_We1,
                            gat_a1[2], gat_We2, gat_a2[2])

    tsrc, sdst = _proj(x, gat_W0, gat_a0[0], gat_a0[1], True)
    acc = _sc_edge(tsrc, sdst, se0, src, dst)
    tsrc, sdst = _proj(acc, gat_W1, gat_a1[0], gat_a1[1], False)
    acc = _sc_edge(tsrc, sdst, se1, src, dst)
    tsrc, sdst = _proj(acc, gat_W2, gat_a2[0], gat_a2[1], False)
    acc = _sc_edge(tsrc, sdst, se2, src, dst)

    hsum = _reduce(acc)
    grows = _sc_gather(acc, valid_node_indices.astype(_i32))

    sb = (steps_till_done / ep_length).reshape(1, 1).astype(_f32)
    q = _final(grows, hsum, sb,
               A_W1[:D], A_W1[D:2 * D], A_W1[2 * D:2 * D + 1],
               A_b1.reshape(1, -1), A_W2, A_b2.reshape(1, 1),
               V_W1[:D], V_W1[D:D + 1], V_b1.reshape(1, -1),
               V_W2, V_b2.reshape(1, 1))
    return q.reshape(1024), valid_node_indices


# speedup vs baseline: 51.4309x; 1.1210x over previous
import functools
import jax
import jax.numpy as jnp
from jax import lax
from jax.experimental import pallas as pl
from jax.experimental.pallas import tpu as pltpu
from jax.experimental.pallas import tpu_sc as plsc

N_NODES = 10000
N_EDGES = 320000
HEADS = 8
HIDDEN = 32
D = HEADS * HIDDEN
NC, NS, LANES = 2, 16, 16
HH = HEADS // NC
HD = HH * HIDDEN
ROW = 144
CHUNK = 112
KCH = 179
E_PAD = NS * KCH * CHUNK
SE_ROWS = E_PAD // LANES
NBLK = 400
EBLK = 2000

_i32 = jnp.int32
_f32 = jnp.float32

_SC_PARAMS = pltpu.CompilerParams(use_tc_tiling_on_sc=False,
                                  needs_layout_passes=False)


def _unsplit(blk):
    feat = jnp.concatenate([blk[0, :, :HD], blk[1, :, :HD]], axis=1)
    denom = jnp.concatenate(
        [blk[0, :, HD:HD + HH], blk[1, :, HD:HD + HH]], axis=1)
    return feat, denom


def _norm_elu(blk):
    n = blk.shape[1]
    feat, denom = _unsplit(blk)
    f3 = feat.reshape(n, HEADS, HIDDEN)
    d3 = denom.reshape(n, HEADS, 1)
    hv = (f3 / (d3 + 1e-16)).reshape(n, D)
    return jnp.where(hv > 0, hv, jnp.exp(jnp.minimum(hv, 0.0)) - 1.0)


def _proj_body(first, h_ref, w_ref, a0_ref, a1_ref, tsrc_ref, sdst_ref):
    if first:
        h = h_ref[...]
    else:
        h = _norm_elu(h_ref[...])
    hp = jnp.dot(h, w_ref[...], preferred_element_type=_f32)
    hp3 = hp.reshape(NBLK, HEADS, HIDDEN)
    s0 = jnp.sum(hp3 * a0_ref[...][None], axis=-1)
    s1 = jnp.sum(hp3 * a1_ref[...][None], axis=-1)
    zpad = jnp.zeros((NBLK, ROW - HD - HH), _f32)
    halves = []
    for c in range(NC):
        halves.append(jnp.concatenate(
            [hp[:, c * HD:(c + 1) * HD], s0[:, c * HH:(c + 1) * HH], zpad],
            axis=1))
    tsrc_ref[...] = jnp.stack(halves, axis=0)
    sdst_ref[...] = jnp.concatenate(
        [s1, jnp.zeros((NBLK, LANES - HEADS), _f32)], axis=1)


def _proj(h, w, a0, a1, first):
    if first:
        hspec = pl.BlockSpec((NBLK, h.shape[1]), lambda i: (i, 0))
        wrows = h.shape[1]
    else:
        hspec = pl.BlockSpec((NC, NBLK, ROW), lambda i: (0, i, 0))
        wrows = D
    return pl.pallas_call(
        functools.partial(_proj_body, first),
        grid=(N_NODES // NBLK,),
        in_specs=[
            hspec,
            pl.BlockSpec((wrows, D), lambda i: (0, 0)),
            pl.BlockSpec((HEADS, HIDDEN), lambda i: (0, 0)),
            pl.BlockSpec((HEADS, HIDDEN), lambda i: (0, 0)),
        ],
        out_specs=[
            pl.BlockSpec((NC, NBLK, ROW), lambda i: (0, i, 0)),
            pl.BlockSpec((NBLK, LANES), lambda i: (i, 0)),
        ],
        out_shape=[
            jax.ShapeDtypeStruct((NC, N_NODES, ROW), _f32),
            jax.ShapeDtypeStruct((N_NODES, LANES), _f32),
        ],
    )(h, w, a0, a1)


def _se_body(ea_ref, we0, a20, we1, a21, we2, a22, o0, o1, o2):
    ea = ea_ref[...]
    for we_ref, a_ref, o_ref in ((we0, a20, o0), (we1, a21, o1), (we2, a22, o2)):
        c = jnp.sum(we_ref[...].reshape(16, HEADS, HIDDEN) * a_ref[...][None],
                    axis=-1)
        se = jnp.dot(ea, c, preferred_element_type=_f32)
        slabs = []
        for cc in range(NC):
            sec = se[:, cc * HH:(cc + 1) * HH]
            slabs.append(sec.reshape(EBLK // LANES, LANES, HH).transpose(0, 2, 1))
        o_ref[...] = jnp.stack(slabs, axis=0)


def _se_all(edge_attr, we0, a20, we1, a21, we2, a22):
    rows = EBLK // LANES
    wspec = pl.BlockSpec((16, D), lambda i: (0, 0))
    aspec = pl.BlockSpec((HEADS, HIDDEN), lambda i: (0, 0))
    ospec = pl.BlockSpec((NC, rows, HH, LANES), lambda i: (0, i, 0, 0))
    oshape = jax.ShapeDtypeStruct((NC, SE_ROWS, HH, LANES), _f32)
    return pl.pallas_call(
        _se_body,
        grid=(N_EDGES // EBLK,),
        in_specs=[pl.BlockSpec((EBLK, 16), lambda i: (i, 0)),
                  wspec, aspec, wspec, aspec, wspec, aspec],
        out_specs=[ospec, ospec, ospec],
        out_shape=[oshape, oshape, oshape],
    )(edge_attr, we0, a20, we1, a21, we2, a22)


def _edge_body(tsrc, sdst, se, ei, out,
               rows0, rows1, sdv0, sdv1, sev0, sev1, eiv0, eiv1,
               dsts0, dsts1, exbuf, zv, acc,
               psem0, psem1, gsem0, gsem1, ssem0, ssem1):
    c = lax.axis_index("c")
    s = lax.axis_index("s")
    zero16 = jnp.zeros((LANES,), _f32)
    iota16 = lax.iota(_i32, LANES)
    rows = (rows0, rows1)
    sdvs = (sdv0, sdv1)
    sevs = (sev0, sev1)
    eivs = (eiv0, eiv1)
    dsts = (dsts0, dsts1)
    psems = (psem0, psem1)
    gsems = (gsem0, gsem1)
    ssems = (ssem0, ssem1)

    def zrow(r, _):
        for k in range(ROW // LANES):
            zv[r, pl.ds(k * LANES, LANES)] = zero16
        return 0
    lax.fori_loop(0, LANES, zrow, 0)

    def zacc(i, _):
        pltpu.sync_copy(zv, acc.at[pl.ds(s * 624 + i * LANES, LANES)])
        return 0
    lax.fori_loop(0, 39, zacc, 0)
    @pl.when(s == NS - 1)
    def _():
        pltpu.sync_copy(zv, acc.at[pl.ds(9984, LANES)])
    plsc.subcore_barrier()

    src_off = c * N_NODES

    def stage1(j, ci):
        gci = ci * NS + s
        pltpu.async_copy(ei.at[:, pl.ds(gci * CHUNK, CHUNK)], eivs[j], psems[j])
        pltpu.async_copy(
            se.at[c].at[pl.ds(gci * (CHUNK // LANES), CHUNK // LANES)],
            sevs[j], psems[j])

    def stage2(j):
        pltpu.make_async_copy(ei.at[:, pl.ds(0, CHUNK)], eivs[j], psems[j]).wait()
        pltpu.make_async_copy(
            se.at[c].at[pl.ds(0, CHUNK // LANES)], sevs[j], psems[j]).wait()
        for g in range(CHUNK // LANES):
            sl = pl.ds(g * LANES, LANES)
            eivs[j][0, sl] = eivs[j][0, sl] + src_off
        pltpu.async_copy(tsrc.at[eivs[j].at[0]], rows[j], gsems[j])
        pltpu.async_copy(sdst.at[eivs[j].at[1]], sdvs[j], gsems[j])

    def waitg(j):
        pltpu.make_async_copy(tsrc.at[eivs[j].at[0]], rows[j], gsems[j]).wait()
        pltpu.make_async_copy(sdst.at[eivs[j].at[1]], sdvs[j], gsems[j]).wait()

    def waits(j):
        pltpu.make_async_copy(rows[j], acc.at[dsts[j]], ssems[j]).wait()

    def compute(j, ci):
        rows_v = rows[j]
        sdv = sdvs[j]
        sev = sevs[j]
        ebase = (ci * NS + s) * CHUNK

        def group(g, _):
            rid = iota16 + g * LANES
            valid = (rid + (ebase + g * LANES)) < N_EDGES
            dsl = pl.ds(g * LANES, LANES)
            dsts[j][dsl] = eivs[j][1, dsl]
            exvecs = []
            for h in range(HH):
                ss = plsc.load_gather(rows_v, [rid, jnp.full((LANES,), HD + h, _i32)])
                sd = plsc.load_gather(sdv, [rid, jnp.full((LANES,), h, _i32) + c * HH])
                lg = ss + sd + sev[g, h, :]
                lg = jnp.where(lg > 0, lg, 0.2 * lg)
                ex = jnp.where(valid, jnp.exp(lg), 0.0)
                exbuf[h, :] = ex
                exvecs.append(ex)
            for e in range(LANES):
                r = g * LANES + e
                for h in range(HH):
                    sc = exvecs[h][e]
                    for q in range(HIDDEN // LANES):
                        fsl = pl.ds(h * HIDDEN + q * LANES, LANES)
                        rows_v[r, fsl] = rows_v[r, fsl] * sc
                exv = plsc.load_gather(
                    exbuf, [iota16 & 3, jnp.full((LANES,), e, _i32)])
                rows_v[r, pl.ds(HD, LANES)] = jnp.where(iota16 < HH, exv, 0.0)
            return 0

        lax.fori_loop(0, CHUNK // LANES, group, 0)
        pltpu.make_async_copy(rows_v, acc.at[dsts[j]], ssems[j]).start(add=True)

    stage1(0, 0)
    stage2(0)
    stage1(1, 1)
    npairs = KCH // 2

    def pair(k, _):
        a = 2 * k
        waitg(0)
        @pl.when(k > 0)
        def _():
            waits(1)
        stage2(1)
        compute(0, a)
        @pl.when(a + 2 < KCH)
        def _():
            stage1(0, a + 2)
        waitg(1)
        waits(0)
        @pl.when(a + 2 < KCH)
        def _():
            stage2(0)
        compute(1, a + 1)
        @pl.when(a + 3 < KCH)
        def _():
            stage1(1, a + 3)
        return 0

    lax.fori_loop(0, npairs, pair, 0)
    waitg(0)
    waits(1)
    compute(0, KCH - 1)
    waits(0)
    plsc.subcore_barrier()

    pltpu.sync_copy(acc.at[pl.ds(s * 624, 624)],
                    out.at[c].at[pl.ds(s * 624, 624)])
    @pl.when(s == NS - 1)
    def _():
        pltpu.sync_copy(acc.at[pl.ds(9984, LANES)],
                        out.at[c].at[pl.ds(9984, LANES)])


def _sc_edge(tsrc, sdst, se, ei):
    mesh = plsc.VectorSubcoreMesh(core_axis_name="c", subcore_axis_name="s")
    f = pl.kernel(
        _edge_body,
        out_type=jax.ShapeDtypeStruct((NC, N_NODES, ROW), _f32),
        mesh=mesh,
        compiler_params=_SC_PARAMS,
        scratch_types=[
            pltpu.VMEM((CHUNK, ROW), _f32),
            pltpu.VMEM((CHUNK, ROW), _f32),
            pltpu.VMEM((CHUNK, LANES), _f32),
            pltpu.VMEM((CHUNK, LANES), _f32),
            pltpu.VMEM((CHUNK // LANES, HH, LANES), _f32),
            pltpu.VMEM((CHUNK // LANES, HH, LANES), _f32),
            pltpu.VMEM((2, CHUNK), _i32),
            pltpu.VMEM((2, CHUNK), _i32),
            pltpu.VMEM((CHUNK,), _i32),
            pltpu.VMEM((CHUNK,), _i32),
            pltpu.VMEM((HH, LANES), _f32),
            pltpu.VMEM((LANES, ROW), _f32),
            pltpu.VMEM_SHARED((N_NODES, ROW), _f32),
            pltpu.SemaphoreType.DMA,
            pltpu.SemaphoreType.DMA,
            pltpu.SemaphoreType.DMA,
            pltpu.SemaphoreType.DMA,
            pltpu.SemaphoreType.DMA,
            pltpu.SemaphoreType.DMA,
        ],
    )
    return f(tsrc.reshape(NC * N_NODES, ROW), sdst, se, ei)


def _gather_body(table, idx, out, idx_v, rows_v, sem):
    c = lax.axis_index("c")
    s = lax.axis_index("s")
    wid = s * NC + c
    bpw = 1024 // (NC * NS)
    base = wid * bpw
    pltpu.sync_copy(idx.at[pl.ds(base, bpw)], idx_v)
    for half in range(NC):
        pltpu.async_copy(table.at[half].at[idx_v], rows_v, sem).wait()
        pltpu.sync_copy(rows_v, out.at[half].at[pl.ds(base, bpw)])


def _sc_gather(table, idx):
    bpw = 1024 // (NC * NS)
    mesh = plsc.VectorSubcoreMesh(core_axis_name="c", subcore_axis_name="s")
    f = pl.kernel(
        _gather_body,
        out_type=jax.ShapeDtypeStruct((NC, 1024, ROW), _f32),
        mesh=mesh,
        compiler_params=_SC_PARAMS,
        scratch_types=[
            pltpu.VMEM((bpw,), _i32),
            pltpu.VMEM((bpw, ROW), _f32),
            pltpu.SemaphoreType.DMA,
        ],
    )
    return f(table, idx)


def _reduce_body(acc_ref, o_ref):
    i = pl.program_id(0)
    h = _norm_elu(acc_ref[...])
    part = jnp.sum(h.reshape(NBLK // 8, 8, D), axis=0)

    @pl.when(i == 0)
    def _():
        o_ref[...] = jnp.zeros((8, D), _f32)
    o_ref[...] += part


def _reduce(acc):
    return pl.pallas_call(
        _reduce_body,
        grid=(N_NODES // NBLK,),
        in_specs=[pl.BlockSpec((NC, NBLK, ROW), lambda i: (0, i, 0))],
        out_specs=pl.BlockSpec((8, D), lambda i: (0, 0)),
        out_shape=jax.ShapeDtypeStruct((8, D), _f32),
    )(acc)


def _final_body(g_ref, hsum_ref, sb_ref, aw1a, aw1b, aw1c, ab1, aw2, ab2,
                vw1a, vw1c, vb1, vw2, vb2, q_ref):
    nef = _norm_elu(g_ref[...])
    agg = jnp.sum(hsum_ref[...], axis=0, keepdims=True) / N_NODES
    sb = sb_ref[0, 0]

    a1 = jnp.dot(nef, aw1a[...], preferred_element_type=_f32)
    a1 = a1 + jnp.dot(agg, aw1b[...], preferred_element_type=_f32)
    a1 = a1 + sb * aw1c[...] + ab1[...]
    a1 = jnp.maximum(a1, 0.0)
    avals = jnp.dot(a1, aw2[...], preferred_element_type=_f32) + ab2[0, 0]

    v1 = jnp.dot(agg, vw1a[...], preferred_element_type=_f32)
    v1 = v1 + sb * vw1c[...] + vb1[...]
    v1 = jnp.maximum(v1, 0.0)
    v = jnp.dot(v1, vw2[...], preferred_element_type=_f32) + vb2[0, 0]

    q = v[0, 0] + avals - jnp.mean(avals)
    q_ref[...] = q.reshape(8, 128)


def _final(grows, hsum, sb, aw1a, aw1b, aw1c, ab1, aw2, ab2,
           vw1a, vw1c, vb1, vw2, vb2):
    args = (grows, hsum, sb, aw1a, aw1b, aw1c, ab1, aw2, ab2,
            vw1a, vw1c, vb1, vw2, vb2)
    specs = []
    for x in args:
        nd = len(x.shape)
        specs.append(pl.BlockSpec(x.shape, (lambda i: (0, 0, 0)) if nd == 3
                                  else (lambda i: (0, 0))))
    return pl.pallas_call(
        _final_body,
        grid=(1,),
        in_specs=specs,
        out_specs=pl.BlockSpec((8, 128), lambda i: (0, 0)),
        out_shape=jax.ShapeDtypeStruct((8, 128), _f32),
    )(*args)


def kernel(x, edge_index, edge_attr, valid_node_indices, steps_till_done,
           ep_length, gat_W0, gat_We0, gat_a0, gat_W1, gat_We1, gat_a1,
           gat_W2, gat_We2, gat_a2, A_W1, A_b1, A_W2, A_b2,
           V_W1, V_b1, V_W2, V_b2):
    ei = jnp.concatenate(
        [edge_index.astype(_i32),
         jnp.zeros((2, E_PAD - N_EDGES), _i32)], axis=1)

    se0, se1, se2 = _se_all(edge_attr, gat_We0, gat_a0[2], gat_We1,
                            gat_a1[2], gat_We2, gat_a2[2])

    tsrc, sdst = _proj(x, gat_W0, gat_a0[0], gat_a0[1], True)
    acc = _sc_edge(tsrc, sdst, se0, ei)
    tsrc, sdst = _proj(acc, gat_W1, gat_a1[0], gat_a1[1], False)
    acc = _sc_edge(tsrc, sdst, se1, ei)
    tsrc, sdst = _proj(acc, gat_W2, gat_a2[0], gat_a2[1], False)
    acc = _sc_edge(tsrc, sdst, se2, ei)

    hsum = _reduce(acc)
    grows = _sc_gather(acc, valid_node_indices.astype(_i32))

    sb = (steps_till_done / ep_length).reshape(1, 1).astype(_f32)
    q = _final(grows, hsum, sb,
               A_W1[:D], A_W1[D:2 * D], A_W1[2 * D:2 * D + 1],
               A_b1.reshape(1, -1), A_W2, A_b2.reshape(1, 1),
               V_W1[:D], V_W1[D:D + 1], V_b1.reshape(1, -1),
               V_W2, V_b2.reshape(1, 1))
    return q.reshape(1024), valid_node_indices


# speedup vs baseline: 60.9006x; 1.1841x over previous
import functools
import jax
import jax.numpy as jnp
from jax import lax
from jax.experimental import pallas as pl
from jax.experimental.pallas import tpu as pltpu
from jax.experimental.pallas import tpu_sc as plsc

N_NODES = 10000
N_EDGES = 320000
HEADS = 8
HIDDEN = 32
D = HEADS * HIDDEN
NC, NS, LANES = 2, 16, 16
HH = HEADS // NC
HD = HH * HIDDEN
ROW = 144
CHUNK = 112
KCH = 179
E_PAD = NS * KCH * CHUNK
SE_ROWS = E_PAD // LANES
NBLK = 400
EBLK = 2000

_i32 = jnp.int32
_f32 = jnp.float32

_SC_PARAMS = pltpu.CompilerParams(use_tc_tiling_on_sc=False,
                                  needs_layout_passes=False)


def _unsplit(blk):
    feat = jnp.concatenate([blk[0, :, :HD], blk[1, :, :HD]], axis=1)
    denom = jnp.concatenate(
        [blk[0, :, HD:HD + HH], blk[1, :, HD:HD + HH]], axis=1)
    return feat, denom


def _norm_elu(blk):
    n = blk.shape[1]
    feat, denom = _unsplit(blk)
    f3 = feat.reshape(n, HEADS, HIDDEN)
    d3 = denom.reshape(n, HEADS, 1)
    hv = (f3 / (d3 + 1e-16)).reshape(n, D)
    return jnp.where(hv > 0, hv, jnp.exp(jnp.minimum(hv, 0.0)) - 1.0)


def _proj_body(first, h_ref, w_ref, a0_ref, a1_ref, tsrc_ref, sdst_ref):
    if first:
        h = h_ref[...]
    else:
        h = _norm_elu(h_ref[...])
    hp = jnp.dot(h, w_ref[...], preferred_element_type=_f32)
    hp3 = hp.reshape(NBLK, HEADS, HIDDEN)
    s0 = jnp.sum(hp3 * a0_ref[...][None], axis=-1)
    s1 = jnp.sum(hp3 * a1_ref[...][None], axis=-1)
    zpad = jnp.zeros((NBLK, ROW - HD - HH), _f32)
    halves = []
    for c in range(NC):
        halves.append(jnp.concatenate(
            [hp[:, c * HD:(c + 1) * HD], s0[:, c * HH:(c + 1) * HH], zpad],
            axis=1))
    tsrc_ref[...] = jnp.stack(halves, axis=0)
    sdst_ref[...] = jnp.concatenate(
        [s1, jnp.zeros((NBLK, LANES - HEADS), _f32)], axis=1)


def _proj(h, w, a0, a1, first):
    if first:
        hspec = pl.BlockSpec((NBLK, h.shape[1]), lambda i: (i, 0))
        wrows = h.shape[1]
    else:
        hspec = pl.BlockSpec((NC, NBLK, ROW), lambda i: (0, i, 0))
        wrows = D
    return pl.pallas_call(
        functools.partial(_proj_body, first),
        grid=(N_NODES // NBLK,),
        in_specs=[
            hspec,
            pl.BlockSpec((wrows, D), lambda i: (0, 0)),
            pl.BlockSpec((HEADS, HIDDEN), lambda i: (0, 0)),
            pl.BlockSpec((HEADS, HIDDEN), lambda i: (0, 0)),
        ],
        out_specs=[
            pl.BlockSpec((NC, NBLK, ROW), lambda i: (0, i, 0)),
            pl.BlockSpec((NBLK, LANES), lambda i: (i, 0)),
        ],
        out_shape=[
            jax.ShapeDtypeStruct((NC, N_NODES, ROW), _f32),
            jax.ShapeDtypeStruct((N_NODES, LANES), _f32),
        ],
    )(h, w, a0, a1)


def _se_body(ea_ref, we0, a20, we1, a21, we2, a22, o0, o1, o2):
    ea = ea_ref[...]
    for we_ref, a_ref, o_ref in ((we0, a20, o0), (we1, a21, o1), (we2, a22, o2)):
        c = jnp.sum(we_ref[...].reshape(16, HEADS, HIDDEN) * a_ref[...][None],
                    axis=-1)
        se = jnp.dot(ea, c, preferred_element_type=_f32)
        slabs = []
        for cc in range(NC):
            sec = se[:, cc * HH:(cc + 1) * HH]
            slabs.append(sec.reshape(EBLK // LANES, LANES, HH).transpose(0, 2, 1))
        o_ref[...] = jnp.stack(slabs, axis=0)


def _se_all(edge_attr, we0, a20, we1, a21, we2, a22):
    rows = EBLK // LANES
    wspec = pl.BlockSpec((16, D), lambda i: (0, 0))
    aspec = pl.BlockSpec((HEADS, HIDDEN), lambda i: (0, 0))
    ospec = pl.BlockSpec((NC, rows, HH, LANES), lambda i: (0, i, 0, 0))
    oshape = jax.ShapeDtypeStruct((NC, SE_ROWS, HH, LANES), _f32)
    return pl.pallas_call(
        _se_body,
        grid=(N_EDGES // EBLK,),
        in_specs=[pl.BlockSpec((EBLK, 16), lambda i: (i, 0)),
                  wspec, aspec, wspec, aspec, wspec, aspec],
        out_specs=[ospec, ospec, ospec],
        out_shape=[oshape, oshape, oshape],
    )(edge_attr, we0, a20, we1, a21, we2, a22)


def _edge_body(tsrc, sdst, se, ei, out,
               rows0, rows1, sdv0, sdv1, sev0, sev1, eiv0, eiv1,
               dsts0, dsts1, zv, acc,
               psem0, psem1, gsem0, gsem1, ssem0, ssem1):
    c = lax.axis_index("c")
    s = lax.axis_index("s")
    zero16 = jnp.zeros((LANES,), _f32)
    iota16 = lax.iota(_i32, LANES)
    rows = (rows0, rows1)
    sdvs = (sdv0, sdv1)
    sevs = (sev0, sev1)
    eivs = (eiv0, eiv1)
    dsts = (dsts0, dsts1)
    psems = (psem0, psem1)
    gsems = (gsem0, gsem1)
    ssems = (ssem0, ssem1)

    def zrow(r, _):
        for k in range(ROW // LANES):
            zv[r, pl.ds(k * LANES, LANES)] = zero16
        return 0
    lax.fori_loop(0, LANES, zrow, 0)

    def zacc(i, _):
        pltpu.sync_copy(zv, acc.at[pl.ds(s * 624 + i * LANES, LANES)])
        return 0
    lax.fori_loop(0, 39, zacc, 0)
    @pl.when(s == NS - 1)
    def _():
        pltpu.sync_copy(zv, acc.at[pl.ds(9984, LANES)])
    plsc.subcore_barrier()

    src_off = c * N_NODES

    def stage1(j, ci):
        gci = ci * NS + s
        pltpu.async_copy(ei.at[:, pl.ds(gci * CHUNK, CHUNK)], eivs[j], psems[j])
        pltpu.async_copy(
            se.at[c].at[pl.ds(gci * (CHUNK // LANES), CHUNK // LANES)],
            sevs[j], psems[j])

    def stage2(j):
        pltpu.make_async_copy(ei.at[:, pl.ds(0, CHUNK)], eivs[j], psems[j]).wait()
        pltpu.make_async_copy(
            se.at[c].at[pl.ds(0, CHUNK // LANES)], sevs[j], psems[j]).wait()
        for g in range(CHUNK // LANES):
            sl = pl.ds(g * LANES, LANES)
            eivs[j][0, sl] = eivs[j][0, sl] + src_off
        pltpu.async_copy(tsrc.at[eivs[j].at[0]], rows[j], gsems[j])
        pltpu.async_copy(sdst.at[eivs[j].at[1]], sdvs[j], gsems[j])

    def waitg(j):
        pltpu.make_async_copy(tsrc.at[eivs[j].at[0]], rows[j], gsems[j]).wait()
        pltpu.make_async_copy(sdst.at[eivs[j].at[1]], sdvs[j], gsems[j]).wait()

    def waits(j):
        pltpu.make_async_copy(rows[j], acc.at[dsts[j]], ssems[j]).wait()

    def compute(j, ci):
        rows_v = rows[j]
        sdv = sdvs[j]
        sev = sevs[j]
        ebase = (ci * NS + s) * CHUNK

        def group(g, _):
            rid = iota16 + g * LANES
            valid = (rid + (ebase + g * LANES)) < N_EDGES
            dsl = pl.ds(g * LANES, LANES)
            dsts[j][dsl] = eivs[j][1, dsl]
            exvecs = []
            for h in range(HH):
                ss = plsc.load_gather(rows_v, [rid, jnp.full((LANES,), HD + h, _i32)])
                sd = plsc.load_gather(sdv, [rid, jnp.full((LANES,), h, _i32) + c * HH])
                lg = ss + sd + sev[g, h, :]
                lg = jnp.where(lg > 0, lg, 0.2 * lg)
                ex = jnp.where(valid, jnp.exp(lg), 0.0)
                exvecs.append(ex)
            for e in range(LANES):
                r = g * LANES + e
                for h in range(HH):
                    sc = exvecs[h][e]
                    for q in range(HIDDEN // LANES):
                        fsl = pl.ds(h * HIDDEN + q * LANES, LANES)
                        rows_v[r, fsl] = rows_v[r, fsl] * sc
            for h in range(HH):
                plsc.store_scatter(
                    rows_v, [rid, jnp.full((LANES,), HD + h, _i32)], exvecs[h])
            return 0

        lax.fori_loop(0, CHUNK // LANES, group, 0)
        pltpu.make_async_copy(rows_v, acc.at[dsts[j]], ssems[j]).start(add=True)

    stage1(0, 0)
    stage2(0)
    stage1(1, 1)
    npairs = KCH // 2

    def pair(k, _):
        a = 2 * k
        waitg(0)
        @pl.when(k > 0)
        def _():
            waits(1)
        stage2(1)
        compute(0, a)
        @pl.when(a + 2 < KCH)
        def _():
            stage1(0, a + 2)
        waitg(1)
        waits(0)
        @pl.when(a + 2 < KCH)
        def _():
            stage2(0)
        compute(1, a + 1)
        @pl.when(a + 3 < KCH)
        def _():
            stage1(1, a + 3)
        return 0

    lax.fori_loop(0, npairs, pair, 0)
    waitg(0)
    waits(1)
    compute(0, KCH - 1)
    waits(0)
    plsc.subcore_barrier()

    pltpu.sync_copy(acc.at[pl.ds(s * 624, 624)],
                    out.at[c].at[pl.ds(s * 624, 624)])
    @pl.when(s == NS - 1)
    def _():
        pltpu.sync_copy(acc.at[pl.ds(9984, LANES)],
                        out.at[c].at[pl.ds(9984, LANES)])


def _sc_edge(tsrc, sdst, se, ei):
    mesh = plsc.VectorSubcoreMesh(core_axis_name="c", subcore_axis_name="s")
    f = pl.kernel(
        _edge_body,
        out_type=jax.ShapeDtypeStruct((NC, N_NODES, ROW), _f32),
        mesh=mesh,
        compiler_params=_SC_PARAMS,
        scratch_types=[
            pltpu.VMEM((CHUNK, ROW), _f32),
            pltpu.VMEM((CHUNK, ROW), _f32),
            pltpu.VMEM((CHUNK, LANES), _f32),
            pltpu.VMEM((CHUNK, LANES), _f32),
            pltpu.VMEM((CHUNK // LANES, HH, LANES), _f32),
            pltpu.VMEM((CHUNK // LANES, HH, LANES), _f32),
            pltpu.VMEM((2, CHUNK), _i32),
            pltpu.VMEM((2, CHUNK), _i32),
            pltpu.VMEM((CHUNK,), _i32),
            pltpu.VMEM((CHUNK,), _i32),
            pltpu.VMEM((LANES, ROW), _f32),
            pltpu.VMEM_SHARED((N_NODES, ROW), _f32),
            pltpu.SemaphoreType.DMA,
            pltpu.SemaphoreType.DMA,
            pltpu.SemaphoreType.DMA,
            pltpu.SemaphoreType.DMA,
            pltpu.SemaphoreType.DMA,
            pltpu.SemaphoreType.DMA,
        ],
    )
    return f(tsrc.reshape(NC * N_NODES, ROW), sdst, se, ei)


def _gather_body(table, idx, out, idx_v, rows_v, sem):
    c = lax.axis_index("c")
    s = lax.axis_index("s")
    wid = s * NC + c
    bpw = 1024 // (NC * NS)
    base = wid * bpw
    pltpu.sync_copy(idx.at[pl.ds(base, bpw)], idx_v)
    for half in range(NC):
        pltpu.async_copy(table.at[half].at[idx_v], rows_v, sem).wait()
        pltpu.sync_copy(rows_v, out.at[half].at[pl.ds(base, bpw)])


def _sc_gather(table, idx):
    bpw = 1024 // (NC * NS)
    mesh = plsc.VectorSubcoreMesh(core_axis_name="c", subcore_axis_name="s")
    f = pl.kernel(
        _gather_body,
        out_type=jax.ShapeDtypeStruct((NC, 1024, ROW), _f32),
        mesh=mesh,
        compiler_params=_SC_PARAMS,
        scratch_types=[
            pltpu.VMEM((bpw,), _i32),
            pltpu.VMEM((bpw, ROW), _f32),
            pltpu.SemaphoreType.DMA,
        ],
    )
    return f(table, idx)


def _reduce_body(acc_ref, o_ref):
    i = pl.program_id(0)
    h = _norm_elu(acc_ref[...])
    part = jnp.sum(h.reshape(NBLK // 8, 8, D), axis=0)

    @pl.when(i == 0)
    def _():
        o_ref[...] = jnp.zeros((8, D), _f32)
    o_ref[...] += part


def _reduce(acc):
    return pl.pallas_call(
        _reduce_body,
        grid=(N_NODES // NBLK,),
        in_specs=[pl.BlockSpec((NC, NBLK, ROW), lambda i: (0, i, 0))],
        out_specs=pl.BlockSpec((8, D), lambda i: (0, 0)),
        out_shape=jax.ShapeDtypeStruct((8, D), _f32),
    )(acc)


def _final_body(g_ref, hsum_ref, sb_ref, aw1a, aw1b, aw1c, ab1, aw2, ab2,
                vw1a, vw1c, vb1, vw2, vb2, q_ref):
    nef = _norm_elu(g_ref[...])
    agg = jnp.sum(hsum_ref[...], axis=0, keepdims=True) / N_NODES
    sb = sb_ref[0, 0]

    a1 = jnp.dot(nef, aw1a[...], preferred_element_type=_f32)
    a1 = a1 + jnp.dot(agg, aw1b[...], preferred_element_type=_f32)
    a1 = a1 + sb * aw1c[...] + ab1[...]
    a1 = jnp.maximum(a1, 0.0)
    avals = jnp.dot(a1, aw2[...], preferred_element_type=_f32) + ab2[0, 0]

    v1 = jnp.dot(agg, vw1a[...], preferred_element_type=_f32)
    v1 = v1 + sb * vw1c[...] + vb1[...]
    v1 = jnp.maximum(v1, 0.0)
    v = jnp.dot(v1, vw2[...], preferred_element_type=_f32) + vb2[0, 0]

    q = v[0, 0] + avals - jnp.mean(avals)
    q_ref[...] = q.reshape(8, 128)


def _final(grows, hsum, sb, aw1a, aw1b, aw1c, ab1, aw2, ab2,
           vw1a, vw1c, vb1, vw2, vb2):
    args = (grows, hsum, sb, aw1a, aw1b, aw1c, ab1, aw2, ab2,
            vw1a, vw1c, vb1, vw2, vb2)
    specs = []
    for x in args:
        nd = len(x.shape)
        specs.append(pl.BlockSpec(x.shape, (lambda i: (0, 0, 0)) if nd == 3
                                  else (lambda i: (0, 0))))
    return pl.pallas_call(
        _final_body,
        grid=(1,),
        in_specs=specs,
        out_specs=pl.BlockSpec((8, 128), lambda i: (0, 0)),
        out_shape=jax.ShapeDtypeStruct((8, 128), _f32),
    )(*args)


def kernel(x, edge_index, edge_attr, valid_node_indices, steps_till_done,
           ep_length, gat_W0, gat_We0, gat_a0, gat_W1, gat_We1, gat_a1,
           gat_W2, gat_We2, gat_a2, A_W1, A_b1, A_W2, A_b2,
           V_W1, V_b1, V_W2, V_b2):
    ei = jnp.concatenate(
        [edge_index.astype(_i32),
         jnp.zeros((2, E_PAD - N_EDGES), _i32)], axis=1)

    se0, se1, se2 = _se_all(edge_attr, gat_We0, gat_a0[2], gat_We1,
                            gat_a1[2], gat_We2, gat_a2[2])

    tsrc, sdst = _proj(x, gat_W0, gat_a0[0], gat_a0[1], True)
    acc = _sc_edge(tsrc, sdst, se0, ei)
    tsrc, sdst = _proj(acc, gat_W1, gat_a1[0], gat_a1[1], False)
    acc = _sc_edge(tsrc, sdst, se1, ei)
    tsrc, sdst = _proj(acc, gat_W2, gat_a2[0], gat_a2[1], False)
    acc = _sc_edge(tsrc, sdst, se2, ei)

    hsum = _reduce(acc)
    grows = _sc_gather(acc, valid_node_indices.astype(_i32))

    sb = (steps_till_done / ep_length).reshape(1, 1).astype(_f32)
    q = _final(grows, hsum, sb,
               A_W1[:D], A_W1[D:2 * D], A_W1[2 * D:2 * D + 1],
               A_b1.reshape(1, -1), A_W2, A_b2.reshape(1, 1),
               V_W1[:D], V_W1[D:D + 1], V_b1.reshape(1, -1),
               V_W2, V_b2.reshape(1, 1))
    return q.reshape(1024), valid_node_indices
